# Initial kernel scaffold; baseline (speedup 1.0000x reference)
import functools

import jax
import jax.numpy as jnp
from jax import lax
from jax.experimental import pallas as pl
from jax.experimental.pallas import tpu as pltpu
from jax.experimental.pallas import tpu_sc as plsc

NC = 2
NSUB = 16
NW = NC * NSUB
CF = 8
CV = 8


def _face_lap_kernel(nf, nvp, d, x_hbm, idxf_hbm, wen_hbm, idxl_hbm, wl_hbm,
                     ewa_hbm, ewb_hbm, nsa_hbm, nsb_hbm, lap_hbm,
                     idxf_v, wf_v, gbuf, ewa_b, ewb_b, nsa_b, nsb_b,
                     idxl_v, wl_v, lapbuf, sem):
    wid = lax.axis_index("s") * NC + lax.axis_index("c")
    dh = d // 2
    ngrp = dh // 16

    fpw = nf // NW
    fbase = wid * fpw

    def face_chunk(c, carry):
        fb = fbase + c * CF
        pltpu.sync_copy(idxf_hbm.at[pl.ds(fb * 9, CF * 9)], idxf_v)
        pltpu.sync_copy(wen_hbm.at[pl.ds(fb * 18, CF * 18)], wf_v)
        pltpu.async_copy(x_hbm.at[idxf_v], gbuf, sem).wait()
        for i in range(CF):
            we = [wf_v[i * 18 + t] for t in range(9)]
            wn = [wf_v[i * 18 + 9 + t] for t in range(9)]
            for h, ebuf, nbuf in ((0, ewa_b, nsa_b), (1, ewb_b, nsb_b)):
                def jbody(j, carry2, i=i, h=h, ebuf=ebuf, nbuf=nbuf,
                          we=we, wn=wn):
                    col = h * dh + j * 16
                    g = [gbuf[i * 9 + t, pl.ds(col, 16)] for t in range(9)]
                    ew = we[0] * g[0]
                    ns = wn[0] * g[0]
                    for t in range(1, 9):
                        ew = ew + we[t] * g[t]
                        ns = ns + wn[t] * g[t]
                    ebuf[i, pl.ds(j * 16, 16)] = ew
                    nbuf[i, pl.ds(j * 16, 16)] = ns
                    return carry2
                lax.fori_loop(0, ngrp, jbody, 0)
        pltpu.sync_copy(ewa_b, ewa_hbm.at[pl.ds(fb, CF)])
        pltpu.sync_copy(ewb_b, ewb_hbm.at[pl.ds(fb, CF)])
        pltpu.sync_copy(nsa_b, nsa_hbm.at[pl.ds(fb, CF)])
        pltpu.sync_copy(nsb_b, nsb_hbm.at[pl.ds(fb, CF)])
        return carry

    lax.fori_loop(0, fpw // CF, face_chunk, 0)

    vpw = nvp // NW
    vbase = wid * vpw
    ngrp_full = d // 16

    def lap_chunk(c, carry):
        vb = vbase + c * CV
        pltpu.sync_copy(idxl_hbm.at[pl.ds(vb * 7, CV * 7)], idxl_v)
        pltpu.sync_copy(wl_hbm.at[pl.ds(vb * 7, CV * 7)], wl_v)
        pltpu.async_copy(x_hbm.at[idxl_v], gbuf.at[pl.ds(0, CV * 7)],
                         sem).wait()
        for i in range(CV):
            wl = [wl_v[i * 7 + t] for t in range(7)]
            def jbody(j, carry2, i=i, wl=wl):
                col = j * 16
                g = [gbuf[i * 7 + t, pl.ds(col, 16)] for t in range(7)]
                acc = wl[0] * g[0]
                for t in range(1, 7):
                    acc = acc + wl[t] * g[t]
                lapbuf[i, pl.ds(col, 16)] = acc
                return carry2
            lax.fori_loop(0, ngrp_full, jbody, 0)
        pltpu.sync_copy(lapbuf, lap_hbm.at[pl.ds(vb, CV)])
        return carry

    lax.fori_loop(0, vpw // CV, lap_chunk, 0)


def _f2v_kernel(nvp, d, ewa_hbm, ewb_hbm, nsa_hbm, nsb_hbm, idxv_hbm, wv_hbm,
                gve_hbm, gvn_hbm,
                idxv_v, wv_v, gea, geb, gna, gnb, gvebuf, gvnbuf, sem):
    wid = lax.axis_index("s") * NC + lax.axis_index("c")
    dh = d // 2
    ngrp = dh // 16
    vpw = nvp // NW
    vbase = wid * vpw

    def chunk(c, carry):
        vb = vbase + c * CV
        pltpu.sync_copy(idxv_hbm.at[pl.ds(vb * 6, CV * 6)], idxv_v)
        pltpu.sync_copy(wv_hbm.at[pl.ds(vb * 6, CV * 6)], wv_v)
        pltpu.async_copy(ewa_hbm.at[idxv_v], gea, sem).wait()
        pltpu.async_copy(ewb_hbm.at[idxv_v], geb, sem).wait()
        pltpu.async_copy(nsa_hbm.at[idxv_v], gna, sem).wait()
        pltpu.async_copy(nsb_hbm.at[idxv_v], gnb, sem).wait()
        for i in range(CV):
            w = [wv_v[i * 6 + t] for t in range(6)]
            for h, ebuf, nbuf in ((0, gea, gna), (1, geb, gnb)):
                def jbody(j, carry2, i=i, h=h, ebuf=ebuf, nbuf=nbuf, w=w):
                    col = j * 16
                    ge = [ebuf[i * 6 + t, pl.ds(col, 16)] for t in range(6)]
                    gn = [nbuf[i * 6 + t, pl.ds(col, 16)] for t in range(6)]
                    ae = w[0] * ge[0]
                    an = w[0] * gn[0]
                    for t in range(1, 6):
                        ae = ae + w[t] * ge[t]
                        an = an + w[t] * gn[t]
                    gvebuf[i, pl.ds(h * dh + col, 16)] = ae
                    gvnbuf[i, pl.ds(h * dh + col, 16)] = an
                    return carry2
                lax.fori_loop(0, ngrp, jbody, 0)
        pltpu.sync_copy(gvebuf, gve_hbm.at[pl.ds(vb, CV)])
        pltpu.sync_copy(gvnbuf, gvn_hbm.at[pl.ds(vb, CV)])
        return carry

    lax.fori_loop(0, vpw // CV, chunk, 0)


def _matmul_kernel(x_ref, lap_ref, gve_ref, gvn_ref, w_ref, b_ref, o_ref):
    acc = jnp.dot(x_ref[...], w_ref[0], preferred_element_type=jnp.float32)
    acc += jnp.dot(lap_ref[...], w_ref[1], preferred_element_type=jnp.float32)
    acc += jnp.dot(gve_ref[...], w_ref[2], preferred_element_type=jnp.float32)
    acc += jnp.dot(gvn_ref[...], w_ref[3], preferred_element_type=jnp.float32)
    o_ref[...] = acc + b_ref[...]


def kernel(input, Gi, Gv, Li, Lv, F2Vi, F2Vv, NS, EW, coeffs, bias):
    Bsz, C, nv = input.shape
    nf = NS.shape[0]
    d = Bsz * C
    dh = d // 2
    c_out = coeffs.shape[0]
    nvp = ((nv + NW * CV - 1) // (NW * CV)) * (NW * CV)

    x = input.transpose(2, 0, 1).reshape(nv, d)
    xp = jnp.pad(x, ((0, nvp - nv), (0, 0)))

    idxf = Gi[1].reshape(3, nf, 3).transpose(1, 0, 2).reshape(-1)
    gvr = Gv.reshape(3, nf, 3).transpose(1, 0, 2)
    we = (gvr * EW[:, :, None]).reshape(nf, 9)
    wn = (gvr * NS[:, :, None]).reshape(nf, 9)
    wen = jnp.concatenate([we, wn], axis=1).reshape(-1)

    idxl = jnp.pad(Li[1].reshape(nv, 7), ((0, nvp - nv), (0, 0))).reshape(-1)
    wl = jnp.pad(Lv.reshape(nv, 7), ((0, nvp - nv), (0, 0))).reshape(-1)
    idxv = jnp.pad(F2Vi[1].reshape(nv, 6), ((0, nvp - nv), (0, 0))).reshape(-1)
    wv = jnp.pad(F2Vv.reshape(nv, 6), ((0, nvp - nv), (0, 0))).reshape(-1)

    mesh = plsc.VectorSubcoreMesh(core_axis_name="c", subcore_axis_name="s",
                                  num_cores=NC, num_subcores=NSUB)
    f32 = jnp.float32

    face_lap = pl.kernel(
        functools.partial(_face_lap_kernel, nf, nvp, d),
        out_type=[
            jax.ShapeDtypeStruct((nf, dh), f32),
            jax.ShapeDtypeStruct((nf, dh), f32),
            jax.ShapeDtypeStruct((nf, dh), f32),
            jax.ShapeDtypeStruct((nf, dh), f32),
            jax.ShapeDtypeStruct((nvp, d), f32),
        ],
        mesh=mesh,
        scratch_types=[
            pltpu.VMEM((CF * 9,), jnp.int32),
            pltpu.VMEM((CF * 18,), f32),
            pltpu.VMEM((CF * 9, d), f32),
            pltpu.VMEM((CF, dh), f32),
            pltpu.VMEM((CF, dh), f32),
            pltpu.VMEM((CF, dh), f32),
            pltpu.VMEM((CF, dh), f32),
            pltpu.VMEM((CV * 7,), jnp.int32),
            pltpu.VMEM((CV * 7,), f32),
            pltpu.VMEM((CV, d), f32),
            pltpu.SemaphoreType.DMA,
        ],
    )
    ewa, ewb, nsa, nsb, lap = face_lap(xp, idxf, wen, idxl, wl)

    f2v = pl.kernel(
        functools.partial(_f2v_kernel, nvp, d),
        out_type=[
            jax.ShapeDtypeStruct((nvp, d), f32),
            jax.ShapeDtypeStruct((nvp, d), f32),
        ],
        mesh=mesh,
        scratch_types=[
            pltpu.VMEM((CV * 6,), jnp.int32),
            pltpu.VMEM((CV * 6,), f32),
            pltpu.VMEM((CV * 6, dh), f32),
            pltpu.VMEM((CV * 6, dh), f32),
            pltpu.VMEM((CV * 6, dh), f32),
            pltpu.VMEM((CV * 6, dh), f32),
            pltpu.VMEM((CV, d), f32),
            pltpu.VMEM((CV, d), f32),
            pltpu.SemaphoreType.DMA,
        ],
    )
    gve, gvn = f2v(ewa, ewb, nsa, nsb, idxv, wv)

    m = nvp * Bsz
    bm = 512
    w4 = coeffs.transpose(2, 1, 0)
    b2 = bias.reshape(1, c_out)
    feats = [a.reshape(m, C) for a in (xp, lap, gve, gvn)]

    out2 = pl.pallas_call(
        _matmul_kernel,
        grid=(m // bm,),
        in_specs=[
            pl.BlockSpec((bm, C), lambda i: (i, 0)),
            pl.BlockSpec((bm, C), lambda i: (i, 0)),
            pl.BlockSpec((bm, C), lambda i: (i, 0)),
            pl.BlockSpec((bm, C), lambda i: (i, 0)),
            pl.BlockSpec((4, C, c_out), lambda i: (0, 0, 0)),
            pl.BlockSpec((1, c_out), lambda i: (0, 0)),
        ],
        out_specs=pl.BlockSpec((bm, c_out), lambda i: (i, 0)),
        out_shape=jax.ShapeDtypeStruct((m, c_out), f32),
    )(*feats, w4, b2)

    return out2[: nv * Bsz].reshape(nv, Bsz, c_out).transpose(1, 2, 0)


# speedup vs baseline: 18.9359x; 18.9359x over previous
import functools

import jax
import jax.numpy as jnp
from jax import lax
from jax.experimental import pallas as pl
from jax.experimental.pallas import tpu as pltpu
from jax.experimental.pallas import tpu_sc as plsc

NC = 2
NSUB = 16
NW = NC * NSUB
CF = 8
CV = 8


def _face_lap_kernel(nf, nvp, d, x_hbm, idxf_hbm, wen_hbm, idxl_hbm, wl_hbm,
                     ewa_hbm, ewb_hbm, nsa_hbm, nsb_hbm, lap_hbm,
                     idxf_v, wf_v, gbuf, ewa_b, ewb_b, nsa_b, nsb_b,
                     idxl_v, wl_v, lapbuf, sem):
    wid = lax.axis_index("s") * NC + lax.axis_index("c")
    dh = d // 2
    ngrp = dh // 16

    fpw = nf // NW
    fbase = wid * fpw

    def face_chunk(c, carry):
        fb = fbase + c * CF
        pltpu.sync_copy(idxf_hbm.at[pl.ds(fb * 9, CF * 9)], idxf_v)
        pltpu.sync_copy(wen_hbm.at[pl.ds(fb * 32, CF * 32)], wf_v)
        pltpu.async_copy(x_hbm.at[idxf_v], gbuf, sem).wait()
        for i in range(CF):
            wev = wf_v[pl.ds(i * 32, 16)]
            wnv = wf_v[pl.ds(i * 32 + 16, 16)]
            we = [wev[t] for t in range(9)]
            wn = [wnv[t] for t in range(9)]
            for h, ebuf, nbuf in ((0, ewa_b, nsa_b), (1, ewb_b, nsb_b)):
                def jbody(j, carry2, i=i, h=h, ebuf=ebuf, nbuf=nbuf,
                          we=we, wn=wn):
                    col = h * dh + j * 16
                    g = [gbuf[i * 9 + t, pl.ds(col, 16)] for t in range(9)]
                    ew = we[0] * g[0]
                    ns = wn[0] * g[0]
                    for t in range(1, 9):
                        ew = ew + we[t] * g[t]
                        ns = ns + wn[t] * g[t]
                    ebuf[i, pl.ds(j * 16, 16)] = ew
                    nbuf[i, pl.ds(j * 16, 16)] = ns
                    return carry2
                lax.fori_loop(0, ngrp, jbody, 0)
        pltpu.sync_copy(ewa_b, ewa_hbm.at[pl.ds(fb, CF)])
        pltpu.sync_copy(ewb_b, ewb_hbm.at[pl.ds(fb, CF)])
        pltpu.sync_copy(nsa_b, nsa_hbm.at[pl.ds(fb, CF)])
        pltpu.sync_copy(nsb_b, nsb_hbm.at[pl.ds(fb, CF)])
        return carry

    lax.fori_loop(0, fpw // CF, face_chunk, 0)

    vpw = nvp // NW
    vbase = wid * vpw
    ngrp_full = d // 16

    def lap_chunk(c, carry):
        vb = vbase + c * CV
        pltpu.sync_copy(idxl_hbm.at[pl.ds(vb * 7, CV * 7)], idxl_v)
        pltpu.sync_copy(wl_hbm.at[pl.ds(vb * 8, CV * 8)],
                        wl_v.at[pl.ds(0, CV * 8)])
        pltpu.async_copy(x_hbm.at[idxl_v], gbuf.at[pl.ds(0, CV * 7)],
                         sem).wait()
        for i in range(CV):
            wlv = wl_v[pl.ds(i * 8, 16)]
            wl = [wlv[t] for t in range(7)]
            def jbody(j, carry2, i=i, wl=wl):
                col = j * 16
                g = [gbuf[i * 7 + t, pl.ds(col, 16)] for t in range(7)]
                acc = wl[0] * g[0]
                for t in range(1, 7):
                    acc = acc + wl[t] * g[t]
                lapbuf[i, pl.ds(col, 16)] = acc
                return carry2
            lax.fori_loop(0, ngrp_full, jbody, 0)
        pltpu.sync_copy(lapbuf, lap_hbm.at[pl.ds(vb, CV)])
        return carry

    lax.fori_loop(0, vpw // CV, lap_chunk, 0)


def _f2v_kernel(nvp, d, ewa_hbm, ewb_hbm, nsa_hbm, nsb_hbm, idxv_hbm, wv_hbm,
                gve_hbm, gvn_hbm,
                idxv_v, wv_v, gea, geb, gna, gnb, gvebuf, gvnbuf, sem):
    wid = lax.axis_index("s") * NC + lax.axis_index("c")
    dh = d // 2
    ngrp = dh // 16
    vpw = nvp // NW
    vbase = wid * vpw

    def chunk(c, carry):
        vb = vbase + c * CV
        pltpu.sync_copy(idxv_hbm.at[pl.ds(vb * 6, CV * 6)], idxv_v)
        pltpu.sync_copy(wv_hbm.at[pl.ds(vb * 8, CV * 8)],
                        wv_v.at[pl.ds(0, CV * 8)])
        pltpu.async_copy(ewa_hbm.at[idxv_v], gea, sem).wait()
        pltpu.async_copy(ewb_hbm.at[idxv_v], geb, sem).wait()
        pltpu.async_copy(nsa_hbm.at[idxv_v], gna, sem).wait()
        pltpu.async_copy(nsb_hbm.at[idxv_v], gnb, sem).wait()
        for i in range(CV):
            wvv = wv_v[pl.ds(i * 8, 16)]
            w = [wvv[t] for t in range(6)]
            for h, ebuf, nbuf in ((0, gea, gna), (1, geb, gnb)):
                def jbody(j, carry2, i=i, h=h, ebuf=ebuf, nbuf=nbuf, w=w):
                    col = j * 16
                    ge = [ebuf[i * 6 + t, pl.ds(col, 16)] for t in range(6)]
                    gn = [nbuf[i * 6 + t, pl.ds(col, 16)] for t in range(6)]
                    ae = w[0] * ge[0]
                    an = w[0] * gn[0]
                    for t in range(1, 6):
                        ae = ae + w[t] * ge[t]
                        an = an + w[t] * gn[t]
                    gvebuf[i, pl.ds(h * dh + col, 16)] = ae
                    gvnbuf[i, pl.ds(h * dh + col, 16)] = an
                    return carry2
                lax.fori_loop(0, ngrp, jbody, 0)
        pltpu.sync_copy(gvebuf, gve_hbm.at[pl.ds(vb, CV)])
        pltpu.sync_copy(gvnbuf, gvn_hbm.at[pl.ds(vb, CV)])
        return carry

    lax.fori_loop(0, vpw // CV, chunk, 0)


def _matmul_kernel(x_ref, lap_ref, gve_ref, gvn_ref, w_ref, b_ref, o_ref):
    acc = jnp.dot(x_ref[...], w_ref[0], preferred_element_type=jnp.float32)
    acc += jnp.dot(lap_ref[...], w_ref[1], preferred_element_type=jnp.float32)
    acc += jnp.dot(gve_ref[...], w_ref[2], preferred_element_type=jnp.float32)
    acc += jnp.dot(gvn_ref[...], w_ref[3], preferred_element_type=jnp.float32)
    o_ref[...] = acc + b_ref[...]


def kernel(input, Gi, Gv, Li, Lv, F2Vi, F2Vv, NS, EW, coeffs, bias):
    Bsz, C, nv = input.shape
    nf = NS.shape[0]
    d = Bsz * C
    dh = d // 2
    c_out = coeffs.shape[0]
    nvp = ((nv + NW * CV - 1) // (NW * CV)) * (NW * CV)

    x = input.transpose(2, 0, 1).reshape(nv, d)
    xp = jnp.pad(x, ((0, nvp - nv), (0, 0)))

    idxf = Gi[1].reshape(3, nf, 3).transpose(1, 0, 2).reshape(-1)
    gvr = Gv.reshape(3, nf, 3).transpose(1, 0, 2)
    we = jnp.pad((gvr * EW[:, :, None]).reshape(nf, 9), ((0, 0), (0, 7)))
    wn = jnp.pad((gvr * NS[:, :, None]).reshape(nf, 9), ((0, 0), (0, 7)))
    wen = jnp.concatenate([we, wn], axis=1).reshape(-1)

    idxl = jnp.pad(Li[1].reshape(nv, 7), ((0, nvp - nv), (0, 0))).reshape(-1)
    wl = jnp.pad(Lv.reshape(nv, 7), ((0, nvp - nv), (0, 1))).reshape(-1)
    idxv = jnp.pad(F2Vi[1].reshape(nv, 6), ((0, nvp - nv), (0, 0))).reshape(-1)
    wv = jnp.pad(F2Vv.reshape(nv, 6), ((0, nvp - nv), (0, 2))).reshape(-1)

    mesh = plsc.VectorSubcoreMesh(core_axis_name="c", subcore_axis_name="s",
                                  num_cores=NC, num_subcores=NSUB)
    f32 = jnp.float32

    face_lap = pl.kernel(
        functools.partial(_face_lap_kernel, nf, nvp, d),
        out_type=[
            jax.ShapeDtypeStruct((nf, dh), f32),
            jax.ShapeDtypeStruct((nf, dh), f32),
            jax.ShapeDtypeStruct((nf, dh), f32),
            jax.ShapeDtypeStruct((nf, dh), f32),
            jax.ShapeDtypeStruct((nvp, d), f32),
        ],
        mesh=mesh,
        scratch_types=[
            pltpu.VMEM((CF * 9,), jnp.int32),
            pltpu.VMEM((CF * 32,), f32),
            pltpu.VMEM((CF * 9, d), f32),
            pltpu.VMEM((CF, dh), f32),
            pltpu.VMEM((CF, dh), f32),
            pltpu.VMEM((CF, dh), f32),
            pltpu.VMEM((CF, dh), f32),
            pltpu.VMEM((CV * 7,), jnp.int32),
            pltpu.VMEM((CV * 8 + 8,), f32),
            pltpu.VMEM((CV, d), f32),
            pltpu.SemaphoreType.DMA,
        ],
    )
    ewa, ewb, nsa, nsb, lap = face_lap(xp, idxf, wen, idxl, wl)

    f2v = pl.kernel(
        functools.partial(_f2v_kernel, nvp, d),
        out_type=[
            jax.ShapeDtypeStruct((nvp, d), f32),
            jax.ShapeDtypeStruct((nvp, d), f32),
        ],
        mesh=mesh,
        scratch_types=[
            pltpu.VMEM((CV * 6,), jnp.int32),
            pltpu.VMEM((CV * 8 + 8,), f32),
            pltpu.VMEM((CV * 6, dh), f32),
            pltpu.VMEM((CV * 6, dh), f32),
            pltpu.VMEM((CV * 6, dh), f32),
            pltpu.VMEM((CV * 6, dh), f32),
            pltpu.VMEM((CV, d), f32),
            pltpu.VMEM((CV, d), f32),
            pltpu.SemaphoreType.DMA,
        ],
    )
    gve, gvn = f2v(ewa, ewb, nsa, nsb, idxv, wv)

    m = nvp * Bsz
    bm = 512
    w4 = coeffs.transpose(2, 1, 0)
    b2 = bias.reshape(1, c_out)
    feats = [a.reshape(m, C) for a in (xp, lap, gve, gvn)]

    out2 = pl.pallas_call(
        _matmul_kernel,
        grid=(m // bm,),
        in_specs=[
            pl.BlockSpec((bm, C), lambda i: (i, 0)),
            pl.BlockSpec((bm, C), lambda i: (i, 0)),
            pl.BlockSpec((bm, C), lambda i: (i, 0)),
            pl.BlockSpec((bm, C), lambda i: (i, 0)),
            pl.BlockSpec((4, C, c_out), lambda i: (0, 0, 0)),
            pl.BlockSpec((1, c_out), lambda i: (0, 0)),
        ],
        out_specs=pl.BlockSpec((bm, c_out), lambda i: (i, 0)),
        out_shape=jax.ShapeDtypeStruct((m, c_out), f32),
    )(*feats, w4, b2)

    return out2[: nv * Bsz].reshape(nv, Bsz, c_out).transpose(1, 2, 0)


# speedup vs baseline: 20.9194x; 1.1048x over previous
import functools

import jax
import jax.numpy as jnp
from jax import lax
from jax.experimental import pallas as pl
from jax.experimental.pallas import tpu as pltpu
from jax.experimental.pallas import tpu_sc as plsc

NC = 2
NSUB = 16
NW = NC * NSUB
CF = 8
CV = 8


def _wsum(gref, rows, col, wts):
    acc = wts[0] * gref[rows[0], pl.ds(col, 16)]
    for t in range(1, len(wts)):
        acc = acc + wts[t] * gref[rows[t], pl.ds(col, 16)]
    return acc


def _face_lap_kernel(nf, nvp, cdim, x0, x1, x2, x3,
                     idxf_hbm, wen_hbm, idxl_hbm, wl_hbm,
                     ew0, ew1, ew2, ew3, ns0, ns1, ns2, ns3, lap_hbm,
                     idxA, idxB, wbuf, wlbuf,
                     g0, g1, g2, g3, eb0, eb1, eb2, eb3, nb0, nb1, nb2, nb3,
                     sg0, sg1, sg2, sg3,
                     se0, se1, se2, se3, sn0, sn1, sn2, sn3):
    wid = lax.axis_index("s") * NC + lax.axis_index("c")
    ngrp = cdim // 16
    xs = (x0, x1, x2, x3)
    ewrefs = (ew0, ew1, ew2, ew3)
    nsrefs = (ns0, ns1, ns2, ns3)
    gbufs = (g0, g1, g2, g3)
    ebufs = (eb0, eb1, eb2, eb3)
    nbufs = (nb0, nb1, nb2, nb3)
    gsems = (sg0, sg1, sg2, sg3)
    esems = (se0, se1, se2, se3)
    nsems = (sn0, sn1, sn2, sn3)

    fpw = nf // NW
    fbase = wid * fpw
    nch = fpw // CF

    def face_compute(b, fb):
        gb, ebuf, nbuf = gbufs[b], ebufs[b], nbufs[b]
        for i in range(CF):
            wev = wbuf[pl.ds(i * 32, 16)]
            wnv = wbuf[pl.ds(i * 32 + 16, 16)]
            we = [wev[t] for t in range(9)]
            wn = [wnv[t] for t in range(9)]
            rows = [i * 9 + t for t in range(9)]
            def jbody(j, carry, i=i, we=we, wn=wn, rows=rows,
                      gb=gb, ebuf=ebuf, nbuf=nbuf):
                col = j * 16
                ebuf[i, pl.ds(col, 16)] = _wsum(gb, rows, col, we)
                nbuf[i, pl.ds(col, 16)] = _wsum(gb, rows, col, wn)
                return carry
            lax.fori_loop(0, ngrp, jbody, 0)
        pltpu.async_copy(ebuf, ewrefs[b].at[pl.ds(fb, CF)], esems[b])
        pltpu.async_copy(nbuf, nsrefs[b].at[pl.ds(fb, CF)], nsems[b])

    def face_gather(b, idxbuf):
        pltpu.async_copy(xs[b].at[idxbuf], gbufs[b], gsems[b])

    def face_wait(b):
        pltpu.make_async_copy(xs[b].at[idxA], gbufs[b], gsems[b]).wait()

    def face_drain_out(b, fb):
        pltpu.make_async_copy(ebufs[b], ewrefs[b].at[pl.ds(fb, CF)],
                              esems[b]).wait()
        pltpu.make_async_copy(nbufs[b], nsrefs[b].at[pl.ds(fb, CF)],
                              nsems[b]).wait()

    pltpu.sync_copy(idxf_hbm.at[pl.ds(fbase * 9, CF * 9)], idxA)
    face_gather(0, idxA)
    face_gather(1, idxA)

    def face_body(c, carry):
        fb = fbase + c * CF
        fbn = jnp.minimum(fb + CF, fbase + (nch - 1) * CF)
        pltpu.sync_copy(idxf_hbm.at[pl.ds(fb * 9, CF * 9)], idxB)
        pltpu.sync_copy(wen_hbm.at[pl.ds(fb * 32, CF * 32)], wbuf)
        face_gather(2, idxB)
        face_gather(3, idxB)
        for b in (0, 1):
            face_wait(b)
            @pl.when(c > 0)
            def _(b=b, fb=fb):
                face_drain_out(b, fb)
            face_compute(b, fb)
        pltpu.sync_copy(idxf_hbm.at[pl.ds(fbn * 9, CF * 9)], idxA)
        face_gather(0, idxA)
        face_gather(1, idxA)
        for b in (2, 3):
            face_wait(b)
            @pl.when(c > 0)
            def _(b=b, fb=fb):
                face_drain_out(b, fb)
            face_compute(b, fb)
        return carry

    lax.fori_loop(0, nch, face_body, 0)
    face_wait(0)
    face_wait(1)
    last_fb = fbase + (nch - 1) * CF
    for b in range(4):
        face_drain_out(b, last_fb)

    vpw = nvp // NW
    vbase = wid * vpw
    nchl = vpw // CV
    nrow = CV * 7

    def lap_compute(b, vb):
        gb, ebuf = gbufs[b], ebufs[b]
        for i in range(CV):
            wlv = wlbuf[pl.ds(i * 8, 16)]
            wl = [wlv[t] for t in range(7)]
            rows = [i * 7 + t for t in range(7)]
            def jbody(j, carry, i=i, wl=wl, rows=rows, gb=gb, ebuf=ebuf):
                col = j * 16
                ebuf[i, pl.ds(col, 16)] = _wsum(gb, rows, col, wl)
                return carry
            lax.fori_loop(0, ngrp, jbody, 0)
        pltpu.async_copy(ebuf, lap_hbm.at[pl.ds(b * nvp + vb, CV)], esems[b])

    def lap_gather(b, idxbuf):
        pltpu.async_copy(xs[b].at[idxbuf.at[pl.ds(0, nrow)]],
                         gbufs[b].at[pl.ds(0, nrow)], gsems[b])

    def lap_wait(b):
        pltpu.make_async_copy(xs[b].at[idxA.at[pl.ds(0, nrow)]],
                              gbufs[b].at[pl.ds(0, nrow)], gsems[b]).wait()

    def lap_drain_out(b, vb):
        pltpu.make_async_copy(ebufs[b], lap_hbm.at[pl.ds(b * nvp + vb, CV)],
                              esems[b]).wait()

    pltpu.sync_copy(idxl_hbm.at[pl.ds(vbase * 7, nrow)], idxA.at[pl.ds(0, nrow)])
    lap_gather(0, idxA)
    lap_gather(1, idxA)

    def lap_body(c, carry):
        vb = vbase + c * CV
        vbn = jnp.minimum(vb + CV, vbase + (nchl - 1) * CV)
        pltpu.sync_copy(idxl_hbm.at[pl.ds(vb * 7, nrow)], idxB.at[pl.ds(0, nrow)])
        pltpu.sync_copy(wl_hbm.at[pl.ds(vb * 8, CV * 8)],
                        wlbuf.at[pl.ds(0, CV * 8)])
        lap_gather(2, idxB)
        lap_gather(3, idxB)
        for b in (0, 1):
            lap_wait(b)
            @pl.when(c > 0)
            def _(b=b, vb=vb):
                lap_drain_out(b, vb)
            lap_compute(b, vb)
        pltpu.sync_copy(idxl_hbm.at[pl.ds(vbn * 7, nrow)], idxA.at[pl.ds(0, nrow)])
        lap_gather(0, idxA)
        lap_gather(1, idxA)
        for b in (2, 3):
            lap_wait(b)
            @pl.when(c > 0)
            def _(b=b, vb=vb):
                lap_drain_out(b, vb)
            lap_compute(b, vb)
        return carry

    lax.fori_loop(0, nchl, lap_body, 0)
    lap_wait(0)
    lap_wait(1)
    last_vb = vbase + (nchl - 1) * CV
    for b in range(4):
        lap_drain_out(b, last_vb)


def _f2v_kernel(nf, nvp, cdim, ew0, ew1, ew2, ew3, ns0, ns1, ns2, ns3,
                idxv_hbm, wv_hbm, gve_hbm, gvn_hbm,
                idxA, idxB, wvbuf,
                ge0, ge1, ge2, ge3, gn0, gn1, gn2, gn3,
                ob0, ob1, ob2, ob3, pb0, pb1, pb2, pb3,
                sge0, sge1, sge2, sge3, sgn0, sgn1, sgn2, sgn3,
                so0, so1, so2, so3, sp0, sp1, sp2, sp3):
    wid = lax.axis_index("s") * NC + lax.axis_index("c")
    ngrp = cdim // 16
    ewrefs = (ew0, ew1, ew2, ew3)
    nsrefs = (ns0, ns1, ns2, ns3)
    gebufs = (ge0, ge1, ge2, ge3)
    gnbufs = (gn0, gn1, gn2, gn3)
    obufs = (ob0, ob1, ob2, ob3)
    pbufs = (pb0, pb1, pb2, pb3)
    gesems = (sge0, sge1, sge2, sge3)
    gnsems = (sgn0, sgn1, sgn2, sgn3)
    osems = (so0, so1, so2, so3)
    psems = (sp0, sp1, sp2, sp3)

    vpw = nvp // NW
    vbase = wid * vpw
    nch = vpw // CV
    nrow = CV * 6

    def gather(b, idxbuf):
        pltpu.async_copy(ewrefs[b].at[idxbuf], gebufs[b], gesems[b])
        pltpu.async_copy(nsrefs[b].at[idxbuf], gnbufs[b], gnsems[b])

    def gwait(b):
        pltpu.make_async_copy(ewrefs[b].at[idxA], gebufs[b], gesems[b]).wait()
        pltpu.make_async_copy(nsrefs[b].at[idxA], gnbufs[b], gnsems[b]).wait()

    def compute(b, vb):
        geb, gnb, ob, pb = gebufs[b], gnbufs[b], obufs[b], pbufs[b]
        for i in range(CV):
            wvv = wvbuf[pl.ds(i * 8, 16)]
            w = [wvv[t] for t in range(6)]
            rows = [i * 6 + t for t in range(6)]
            def jbody(j, carry, i=i, w=w, rows=rows, geb=geb, gnb=gnb,
                      ob=ob, pb=pb):
                col = j * 16
                ob[i, pl.ds(col, 16)] = _wsum(geb, rows, col, w)
                pb[i, pl.ds(col, 16)] = _wsum(gnb, rows, col, w)
                return carry
            lax.fori_loop(0, ngrp, jbody, 0)
        pltpu.async_copy(ob, gve_hbm.at[pl.ds(b * nvp + vb, CV)], osems[b])
        pltpu.async_copy(pb, gvn_hbm.at[pl.ds(b * nvp + vb, CV)], psems[b])

    def drain_out(b, vb):
        pltpu.make_async_copy(obufs[b], gve_hbm.at[pl.ds(b * nvp + vb, CV)],
                              osems[b]).wait()
        pltpu.make_async_copy(pbufs[b], gvn_hbm.at[pl.ds(b * nvp + vb, CV)],
                              psems[b]).wait()

    pltpu.sync_copy(idxv_hbm.at[pl.ds(vbase * 6, nrow)], idxA)
    gather(0, idxA)
    gather(1, idxA)

    def body(c, carry):
        vb = vbase + c * CV
        vbn = jnp.minimum(vb + CV, vbase + (nch - 1) * CV)
        pltpu.sync_copy(idxv_hbm.at[pl.ds(vb * 6, nrow)], idxB)
        pltpu.sync_copy(wv_hbm.at[pl.ds(vb * 8, CV * 8)],
                        wvbuf.at[pl.ds(0, CV * 8)])
        gather(2, idxB)
        gather(3, idxB)
        for b in (0, 1):
            gwait(b)
            @pl.when(c > 0)
            def _(b=b, vb=vb):
                drain_out(b, vb)
            compute(b, vb)
        pltpu.sync_copy(idxv_hbm.at[pl.ds(vbn * 6, nrow)], idxA)
        gather(0, idxA)
        gather(1, idxA)
        for b in (2, 3):
            gwait(b)
            @pl.when(c > 0)
            def _(b=b, vb=vb):
                drain_out(b, vb)
            compute(b, vb)
        return carry

    lax.fori_loop(0, nch, body, 0)
    gwait(0)
    gwait(1)
    last_vb = vbase + (nch - 1) * CV
    for b in range(4):
        drain_out(b, last_vb)


def _matmul_kernel(x_ref, lap_ref, gve_ref, gvn_ref, w_ref, b_ref, o_ref):
    acc = jnp.dot(x_ref[...], w_ref[0], preferred_element_type=jnp.float32)
    acc += jnp.dot(lap_ref[...], w_ref[1], preferred_element_type=jnp.float32)
    acc += jnp.dot(gve_ref[...], w_ref[2], preferred_element_type=jnp.float32)
    acc += jnp.dot(gvn_ref[...], w_ref[3], preferred_element_type=jnp.float32)
    o_ref[...] = acc + b_ref[...]


def kernel(input, Gi, Gv, Li, Lv, F2Vi, F2Vv, NS, EW, coeffs, bias):
    Bsz, C, nv = input.shape
    nf = NS.shape[0]
    c_out = coeffs.shape[0]
    nvp = ((nv + NW * CV - 1) // (NW * CV)) * (NW * CV)
    f32 = jnp.float32

    xbn = input.transpose(0, 2, 1)
    xpad = jnp.pad(xbn, ((0, 0), (0, nvp - nv), (0, 0)))
    xs = [xpad[b] for b in range(Bsz)]
    x4 = xpad.reshape(Bsz * nvp, C)

    idxf = Gi[1].reshape(3, nf, 3).transpose(1, 0, 2).reshape(-1)
    gvr = Gv.reshape(3, nf, 3).transpose(1, 0, 2)
    we = jnp.pad((gvr * EW[:, :, None]).reshape(nf, 9), ((0, 0), (0, 7)))
    wn = jnp.pad((gvr * NS[:, :, None]).reshape(nf, 9), ((0, 0), (0, 7)))
    wen = jnp.concatenate([we, wn], axis=1).reshape(-1)

    idxl = jnp.pad(Li[1].reshape(nv, 7), ((0, nvp - nv), (0, 0))).reshape(-1)
    wl = jnp.pad(Lv.reshape(nv, 7), ((0, nvp - nv), (0, 1))).reshape(-1)
    idxv = jnp.pad(F2Vi[1].reshape(nv, 6), ((0, nvp - nv), (0, 0))).reshape(-1)
    wv = jnp.pad(F2Vv.reshape(nv, 6), ((0, nvp - nv), (0, 2))).reshape(-1)

    mesh = plsc.VectorSubcoreMesh(core_axis_name="c", subcore_axis_name="s",
                                  num_cores=NC, num_subcores=NSUB)

    face_lap = pl.kernel(
        functools.partial(_face_lap_kernel, nf, nvp, C),
        out_type=(
            [jax.ShapeDtypeStruct((nf, C), f32) for _ in range(8)]
            + [jax.ShapeDtypeStruct((Bsz * nvp, C), f32)]
        ),
        mesh=mesh,
        scratch_types=(
            [pltpu.VMEM((CF * 9,), jnp.int32)] * 2
            + [pltpu.VMEM((CF * 32,), f32)]
            + [pltpu.VMEM((CV * 8 + 8,), f32)]
            + [pltpu.VMEM((CF * 9, C), f32)] * 4
            + [pltpu.VMEM((CF, C), f32)] * 8
            + [pltpu.SemaphoreType.DMA] * 12
        ),
    )
    outs = face_lap(xs[0], xs[1], xs[2], xs[3], idxf, wen, idxl, wl)
    ewf = outs[0:4]
    nsf = outs[4:8]
    lap = outs[8]

    f2v = pl.kernel(
        functools.partial(_f2v_kernel, nf, nvp, C),
        out_type=[
            jax.ShapeDtypeStruct((Bsz * nvp, C), f32),
            jax.ShapeDtypeStruct((Bsz * nvp, C), f32),
        ],
        mesh=mesh,
        scratch_types=(
            [pltpu.VMEM((CV * 6,), jnp.int32)] * 2
            + [pltpu.VMEM((CV * 8 + 8,), f32)]
            + [pltpu.VMEM((CV * 6, C), f32)] * 8
            + [pltpu.VMEM((CV, C), f32)] * 8
            + [pltpu.SemaphoreType.DMA] * 16
        ),
    )
    gve, gvn = f2v(ewf[0], ewf[1], ewf[2], ewf[3],
                   nsf[0], nsf[1], nsf[2], nsf[3], idxv, wv)

    m = Bsz * nvp
    bm = 512
    w4 = coeffs.transpose(2, 1, 0)
    b2 = bias.reshape(1, c_out)

    out2 = pl.pallas_call(
        _matmul_kernel,
        grid=(m // bm,),
        in_specs=[
            pl.BlockSpec((bm, C), lambda i: (i, 0)),
            pl.BlockSpec((bm, C), lambda i: (i, 0)),
            pl.BlockSpec((bm, C), lambda i: (i, 0)),
            pl.BlockSpec((bm, C), lambda i: (i, 0)),
            pl.BlockSpec((4, C, c_out), lambda i: (0, 0, 0)),
            pl.BlockSpec((1, c_out), lambda i: (0, 0)),
        ],
        out_specs=pl.BlockSpec((bm, c_out), lambda i: (i, 0)),
        out_shape=jax.ShapeDtypeStruct((m, c_out), f32),
    )(x4, lap, gve, gvn, w4, b2)

    return out2.reshape(Bsz, nvp, c_out)[:, :nv].transpose(0, 2, 1)
